# baseline (speedup 1.0000x reference)
import functools

import jax
import jax.numpy as jnp
from jax import lax
from jax.experimental import pallas as pl
from jax.experimental.pallas import tpu as pltpu
from jax.experimental.pallas import tpu_sc as plsc

D = 64
K = 1024
N = 36864
GAMMA = 0.99

NC = 2
NS = 16
NW = NC * NS
CHUNK = N // NW
SUB = 128
NSUB = CHUNK // SUB
ROWS_PER_SUB = K // NS
CNT_W = 16

BLK = 512


def _argmin_body(x_ref, vqt_ref, idx_ref, vqn_ref):
    @pl.when(pl.program_id(0) == 0)
    def _():
        v = vqt_ref[...]
        vqn_ref[...] = jnp.sum(v * v, axis=0, keepdims=True)

    x = x_ref[...]
    xn = jnp.sum(x * x, axis=1, keepdims=True)
    dot = jnp.dot(x, vqt_ref[...], preferred_element_type=jnp.float32)
    d = (xn - 2.0 * dot) + vqn_ref[...]
    m = jnp.min(d, axis=1, keepdims=True)
    col = lax.broadcasted_iota(jnp.int32, d.shape, 1)
    idx_ref[...] = jnp.min(jnp.where(d == m, col, K), axis=1).astype(jnp.int32)


def _compute_indices(x, vqt):
    return pl.pallas_call(
        _argmin_body,
        grid=(N // BLK,),
        in_specs=[
            pl.BlockSpec((BLK, D), lambda i: (i, 0)),
            pl.BlockSpec((D, K), lambda i: (0, 0)),
        ],
        out_specs=pl.BlockSpec((BLK,), lambda i: (i,)),
        out_shape=jax.ShapeDtypeStruct((N,), jnp.int32),
        scratch_shapes=[pltpu.VMEM((1, K), jnp.float32)],
    )(x, vqt)


def _sc_body(x_hbm, idx_hbm, vq_hbm, quant_hbm, psum_hbm, pcnt_hbm,
             idx_v, rows_v, ones_v, zrow_v, zcnt_v, sumtab, cnttab, sem):
    cid = lax.axis_index("c")
    sid = lax.axis_index("s")
    wid = sid * NC + cid
    base = wid * CHUNK

    zero16 = jnp.zeros((16,), jnp.float32)
    one0 = jnp.where(lax.iota(jnp.int32, 16) == 0, 1.0, 0.0).astype(jnp.float32)

    for r in range(16):
        for c in range(D // 16):
            zrow_v[r, pl.ds(c * 16, 16)] = zero16
    for r in range(ROWS_PER_SUB):
        zcnt_v[r, :] = zero16
    for r in range(SUB):
        ones_v[r, :] = one0

    for b in range(ROWS_PER_SUB // 16):
        pltpu.sync_copy(zrow_v, sumtab.at[pl.ds(sid * ROWS_PER_SUB + b * 16, 16)])
    pltpu.sync_copy(zcnt_v, cnttab.at[pl.ds(sid * ROWS_PER_SUB, ROWS_PER_SUB)])

    pltpu.sync_copy(idx_hbm.at[wid], idx_v)

    copies = [
        pltpu.async_copy(vq_hbm.at[idx_v.at[j]],
                         rows_v.at[pl.ds(j * SUB, SUB)], sem)
        for j in range(NSUB)
    ]
    for cp in copies:
        cp.wait()
    pltpu.sync_copy(rows_v, quant_hbm.at[pl.ds(base, CHUNK)])

    plsc.subcore_barrier()

    pltpu.sync_copy(x_hbm.at[pl.ds(base, CHUNK)], rows_v)
    for j in range(NSUB):
        pltpu.sync_copy(rows_v.at[pl.ds(j * SUB, SUB)],
                        sumtab.at[idx_v.at[j]], add=True)
        pltpu.sync_copy(ones_v, cnttab.at[idx_v.at[j]], add=True)

    plsc.subcore_barrier()

    pltpu.sync_copy(sumtab.at[pl.ds(sid * ROWS_PER_SUB, ROWS_PER_SUB)],
                    psum_hbm.at[cid, pl.ds(sid * ROWS_PER_SUB, ROWS_PER_SUB)])
    pltpu.sync_copy(cnttab.at[pl.ds(sid * ROWS_PER_SUB, ROWS_PER_SUB)],
                    pcnt_hbm.at[cid, pl.ds(sid * ROWS_PER_SUB, ROWS_PER_SUB)])


@functools.cache
def _sc_stage():
    return functools.partial(
        pl.kernel,
        out_type=[
            jax.ShapeDtypeStruct((N, D), jnp.float32),
            jax.ShapeDtypeStruct((NC, K, D), jnp.float32),
            jax.ShapeDtypeStruct((NC, K, CNT_W), jnp.float32),
        ],
        mesh=plsc.VectorSubcoreMesh(core_axis_name="c", subcore_axis_name="s",
                                    num_cores=NC, num_subcores=NS),
        compiler_params=pltpu.CompilerParams(use_tc_tiling_on_sc=False),
        scratch_types=[
            pltpu.VMEM((NSUB, SUB), jnp.int32),
            pltpu.VMEM((CHUNK, D), jnp.float32),
            pltpu.VMEM((SUB, CNT_W), jnp.float32),
            pltpu.VMEM((16, D), jnp.float32),
            pltpu.VMEM((ROWS_PER_SUB, CNT_W), jnp.float32),
            pltpu.VMEM_SHARED((K, D), jnp.float32),
            pltpu.VMEM_SHARED((K, CNT_W), jnp.float32),
            pltpu.SemaphoreType.DMA,
        ],
    )(_sc_body)


def _combine_body(es_ref, en_ref, ps_ref, pc_ref, vq_ref, ns_ref, nn_ref):
    csum = ps_ref[0] + ps_ref[1]
    cnt = jnp.sum(pc_ref[0], axis=1) + jnp.sum(pc_ref[1], axis=1)
    new_sum = es_ref[...] * GAMMA + csum * (1.0 - GAMMA)
    new_n = en_ref[...] * GAMMA + cnt * (1.0 - GAMMA)
    ns_ref[...] = new_sum
    nn_ref[...] = new_n
    vq_ref[...] = new_sum / new_n[:, None]


def _combine(ewma_sum, ewma_n, psum, pcnt):
    return pl.pallas_call(
        _combine_body,
        out_shape=[
            jax.ShapeDtypeStruct((K, D), jnp.float32),
            jax.ShapeDtypeStruct((K, D), jnp.float32),
            jax.ShapeDtypeStruct((K,), jnp.float32),
        ],
    )(ewma_sum, ewma_n, psum, pcnt)


def kernel(x, vq, ewma_centroid_sum, ewma_centroid_n):
    idx = _compute_indices(x, jnp.swapaxes(vq, 0, 1))
    quantized, psum, pcnt = _sc_stage()(x, idx.reshape(NW, NSUB, SUB), vq)
    new_vq, new_sum, new_n = _combine(ewma_centroid_sum, ewma_centroid_n,
                                      psum, pcnt)
    return quantized, new_vq, new_sum, new_n


# speedup vs baseline: 1.3762x; 1.3762x over previous
import functools

import jax
import jax.numpy as jnp
from jax import lax
from jax.experimental import pallas as pl
from jax.experimental.pallas import tpu as pltpu
from jax.experimental.pallas import tpu_sc as plsc

D = 64
K = 1024
N = 36864
GAMMA = 0.99

NC = 2
NS = 16
NW = NC * NS
CHUNK = N // NW
SUB = 128
NSUB = CHUNK // SUB

BLK = 6144
SBLK = 6144


def _argmin_body(x_ref, vq_ref, idx_ref, v2_ref, vqn_ref):
    @pl.when(pl.program_id(0) == 0)
    def _():
        v = vq_ref[...]
        v2_ref[...] = v + v
        vqn_ref[...] = jnp.sum(v * v, axis=1, keepdims=True)

    dot2 = lax.dot_general(v2_ref[...], x_ref[...],
                           (((1,), (1,)), ((), ())),
                           preferred_element_type=jnp.float32)
    idx_ref[...] = jnp.argmin(vqn_ref[...] - dot2, axis=0).astype(jnp.int32)


def _compute_indices(x, vq):
    return pl.pallas_call(
        _argmin_body,
        grid=(N // BLK,),
        in_specs=[
            pl.BlockSpec((BLK, D), lambda i: (i, 0)),
            pl.BlockSpec((K, D), lambda i: (0, 0)),
        ],
        out_specs=pl.BlockSpec((BLK,), lambda i: (i,)),
        out_shape=jax.ShapeDtypeStruct((N,), jnp.int32),
        scratch_shapes=[pltpu.VMEM((K, D), jnp.float32),
                        pltpu.VMEM((K, 1), jnp.float32)],
    )(x, vq)


def _sc_body(idx_hbm, vqp_hbm, quant_hbm, idx_v, g0, g1, sem_g, sem_w):
    cid = lax.axis_index("c")
    sid = lax.axis_index("s")
    wid = sid * NC + cid
    base = wid * CHUNK

    pltpu.sync_copy(idx_hbm.at[wid], idx_v)

    bufs = [g0, g1]
    writes = []
    cp = pltpu.async_copy(vqp_hbm.at[idx_v.at[0]], bufs[0], sem_g)
    for j in range(NSUB):
        nxt = None
        if j + 1 < NSUB:
            nxt = pltpu.async_copy(vqp_hbm.at[idx_v.at[j + 1]],
                                   bufs[(j + 1) % 2], sem_g)
        cp.wait()
        writes.append(
            pltpu.async_copy(bufs[j % 2],
                             quant_hbm.at[pl.ds(base + j * SUB, SUB)], sem_w))
        cp = nxt
    for w in writes:
        w.wait()


@functools.cache
def _sc_stage():
    return functools.partial(
        pl.kernel,
        out_type=jax.ShapeDtypeStruct((N, 2 * D), jnp.float32),
        mesh=plsc.VectorSubcoreMesh(core_axis_name="c", subcore_axis_name="s",
                                    num_cores=NC, num_subcores=NS),
        scratch_types=[
            pltpu.VMEM((NSUB, SUB), jnp.int32),
            pltpu.VMEM((SUB, 2 * D), jnp.float32),
            pltpu.VMEM((SUB, 2 * D), jnp.float32),
            pltpu.SemaphoreType.DMA,
            pltpu.SemaphoreType.DMA,
        ],
    )(_sc_body)


def _stats_body(x_ref, idx_ref, es_ref, en_ref,
                vq_ref, ns_ref, nn_ref, macc_ref):
    i = pl.program_id(0)

    @pl.when(i == 0)
    def _():
        macc_ref[...] = jnp.zeros_like(macc_ref)

    idx = idx_ref[...]
    rows = lax.broadcasted_iota(jnp.int32, (K, SBLK), 0)
    mask = (rows == idx[None, :]).astype(jnp.float32)
    x = x_ref[...]
    col = lax.broadcasted_iota(jnp.int32, (SBLK, 2 * D), 1)
    xa = jnp.where(col < D, jnp.pad(x, ((0, 0), (0, D))),
                   jnp.where(col == D, 1.0, 0.0))
    macc_ref[...] += lax.dot_general(mask, xa, (((1,), (0,)), ((), ())),
                                     preferred_element_type=jnp.float32)

    @pl.when(i == pl.num_programs(0) - 1)
    def _():
        macc = macc_ref[...]
        csum = macc[:, :D]
        cnt = jnp.sum(macc[:, D:], axis=1)
        new_sum = es_ref[...] * GAMMA + csum * (1.0 - GAMMA)
        new_n = en_ref[...] * GAMMA + cnt * (1.0 - GAMMA)
        ns_ref[...] = new_sum
        nn_ref[...] = new_n
        vq_ref[...] = new_sum / new_n[:, None]


def _stats(x, idx, ewma_sum, ewma_n):
    return pl.pallas_call(
        _stats_body,
        grid=(N // SBLK,),
        in_specs=[
            pl.BlockSpec((SBLK, D), lambda i: (i, 0)),
            pl.BlockSpec((SBLK,), lambda i: (i,)),
            pl.BlockSpec((K, D), lambda i: (0, 0)),
            pl.BlockSpec((K,), lambda i: (0,)),
        ],
        out_specs=[
            pl.BlockSpec((K, D), lambda i: (0, 0)),
            pl.BlockSpec((K, D), lambda i: (0, 0)),
            pl.BlockSpec((K,), lambda i: (0,)),
        ],
        out_shape=[
            jax.ShapeDtypeStruct((K, D), jnp.float32),
            jax.ShapeDtypeStruct((K, D), jnp.float32),
            jax.ShapeDtypeStruct((K,), jnp.float32),
        ],
        scratch_shapes=[pltpu.VMEM((K, 2 * D), jnp.float32)],
    )(x, idx, ewma_sum, ewma_n)


def kernel(x, vq, ewma_centroid_sum, ewma_centroid_n):
    idx = _compute_indices(x, vq)
    vqp = jnp.concatenate([vq, jnp.zeros((K, D), vq.dtype)], axis=1)
    qpad = _sc_stage()(idx.reshape(NW, NSUB, SUB), vqp)
    new_vq, new_sum, new_n = _stats(x, idx, ewma_centroid_sum,
                                    ewma_centroid_n)
    quantized = qpad[:, :D]
    return quantized, new_vq, new_sum, new_n


# speedup vs baseline: 1.5713x; 1.1418x over previous
import functools

import jax
import jax.numpy as jnp
from jax import lax
from jax.experimental import pallas as pl
from jax.experimental.pallas import tpu as pltpu
from jax.experimental.pallas import tpu_sc as plsc

D = 64
K = 1024
N = 36864
GAMMA = 0.99

NC = 2
NS = 16
NW = NC * NS
CHUNK = N // NW
SUB = 128
NSUB = CHUNK // SUB
NBUF = 4

BLK = 6144
AUG = 8


def _main_body(xt_ref, vqt_ref, es_ref, en_ref,
               idx_ref, vq_ref, ns_ref, nn_ref, v2_ref, vqn_ref, macc_ref):
    i = pl.program_id(0)

    @pl.when(i == 0)
    def _():
        v2 = jnp.swapaxes(vqt_ref[...], 0, 1)
        v2 = v2 + v2
        v2_ref[...] = v2
        vqn_ref[...] = 0.25 * jnp.sum(v2 * v2, axis=1, keepdims=True)
        macc_ref[...] = jnp.zeros_like(macc_ref)

    xt = xt_ref[...]
    dot2 = lax.dot_general(v2_ref[...], xt, (((1,), (0,)), ((), ())),
                           preferred_element_type=jnp.float32)
    idx = jnp.argmin(vqn_ref[...] - dot2, axis=0).astype(jnp.int32)
    idx_ref[...] = idx

    rows = lax.broadcasted_iota(jnp.int32, (K, BLK), 0)
    mask = (rows == idx[None, :]).astype(jnp.float32)
    aug = (lax.broadcasted_iota(jnp.int32, (AUG, BLK), 0) == 0).astype(
        jnp.float32)
    xa_t = jnp.concatenate([xt, aug], axis=0)
    macc_ref[...] += lax.dot_general(mask, xa_t, (((1,), (1,)), ((), ())),
                                     preferred_element_type=jnp.float32)

    @pl.when(i == pl.num_programs(0) - 1)
    def _():
        macc = macc_ref[...]
        csum = macc[:, :D]
        cnt = jnp.sum(macc[:, D:], axis=1)
        new_sum = es_ref[...] * GAMMA + csum * (1.0 - GAMMA)
        new_n = en_ref[...] * GAMMA + cnt * (1.0 - GAMMA)
        ns_ref[...] = new_sum
        nn_ref[...] = new_n
        vq_ref[...] = new_sum / new_n[:, None]


def _main(xt, vqt, ewma_sum, ewma_n):
    return pl.pallas_call(
        _main_body,
        grid=(N // BLK,),
        in_specs=[
            pl.BlockSpec((D, BLK), lambda i: (0, i)),
            pl.BlockSpec((D, K), lambda i: (0, 0)),
            pl.BlockSpec((K, D), lambda i: (0, 0)),
            pl.BlockSpec((K,), lambda i: (0,)),
        ],
        out_specs=[
            pl.BlockSpec((BLK,), lambda i: (i,)),
            pl.BlockSpec((K, D), lambda i: (0, 0)),
            pl.BlockSpec((K, D), lambda i: (0, 0)),
            pl.BlockSpec((K,), lambda i: (0,)),
        ],
        out_shape=[
            jax.ShapeDtypeStruct((N,), jnp.int32),
            jax.ShapeDtypeStruct((K, D), jnp.float32),
            jax.ShapeDtypeStruct((K, D), jnp.float32),
            jax.ShapeDtypeStruct((K,), jnp.float32),
        ],
        scratch_shapes=[pltpu.VMEM((K, D), jnp.float32),
                        pltpu.VMEM((K, 1), jnp.float32),
                        pltpu.VMEM((K, D + AUG), jnp.float32)],
    )(xt, vqt, ewma_sum, ewma_n)


def _sc_body(idx_hbm, vqp_hbm, quant_hbm, idx_v, g0, g1, g2, g3,
             sem_g, sem_w):
    cid = lax.axis_index("c")
    sid = lax.axis_index("s")
    wid = sid * NC + cid
    base = wid * CHUNK

    pltpu.sync_copy(idx_hbm.at[wid], idx_v)

    bufs = [g0, g1, g2, g3]
    gets = [pltpu.async_copy(vqp_hbm.at[idx_v.at[j]], bufs[j % NBUF], sem_g)
            for j in range(min(NBUF, NSUB))]
    gets += [None] * (NSUB - len(gets))
    writes = []
    for j in range(NSUB):
        gets[j].wait()
        writes.append(
            pltpu.async_copy(bufs[j % NBUF],
                             quant_hbm.at[pl.ds(base + j * SUB, SUB)], sem_w))
        nxt = j + NBUF
        if nxt < NSUB:
            if writes and nxt - NBUF >= 0:
                writes[nxt - NBUF].wait()
                writes[nxt - NBUF] = None
            gets[nxt] = pltpu.async_copy(vqp_hbm.at[idx_v.at[nxt]],
                                         bufs[nxt % NBUF], sem_g)
    for w in writes:
        if w is not None:
            w.wait()


@functools.cache
def _sc_stage():
    return functools.partial(
        pl.kernel,
        out_type=jax.ShapeDtypeStruct((N, 2 * D), jnp.float32),
        mesh=plsc.VectorSubcoreMesh(core_axis_name="c", subcore_axis_name="s",
                                    num_cores=NC, num_subcores=NS),
        scratch_types=[
            pltpu.VMEM((NSUB, SUB), jnp.int32),
            pltpu.VMEM((SUB, 2 * D), jnp.float32),
            pltpu.VMEM((SUB, 2 * D), jnp.float32),
            pltpu.VMEM((SUB, 2 * D), jnp.float32),
            pltpu.VMEM((SUB, 2 * D), jnp.float32),
            pltpu.SemaphoreType.DMA,
            pltpu.SemaphoreType.DMA,
        ],
    )(_sc_body)


def kernel(x, vq, ewma_centroid_sum, ewma_centroid_n):
    xt = jnp.swapaxes(x, 0, 1)
    vqt = jnp.swapaxes(vq, 0, 1)
    idx, new_vq, new_sum, new_n = _main(xt, vqt, ewma_centroid_sum,
                                        ewma_centroid_n)
    vqp = jnp.concatenate([vq, jnp.zeros((K, D), vq.dtype)], axis=1)
    qpad = _sc_stage()(idx.reshape(NW, NSUB, SUB), vqp)
    quantized = qpad[:, :D]
    return quantized, new_vq, new_sum, new_n


# speedup vs baseline: 2.3489x; 1.4949x over previous
import functools

import jax
import jax.numpy as jnp
from jax import lax
from jax.experimental import pallas as pl
from jax.experimental.pallas import tpu as pltpu
from jax.experimental.pallas import tpu_sc as plsc

D = 64
K = 1024
N = 36864
GAMMA = 0.99

NC = 2
NS = 16
NW = NC * NS
CHUNK = N // NW
SUB = 128
NSUB = CHUNK // SUB
NBUF = 4

BLK = 6144
AUG = 8


def _main_body(xt_ref, vqt_ref, es_ref, en_ref,
               idx_ref, vq_ref, ns_ref, nn_ref, v2_ref, vqn_ref, macc_ref):
    i = pl.program_id(0)

    @pl.when(i == 0)
    def _():
        v2 = jnp.swapaxes(vqt_ref[...], 0, 1)
        v2 = v2 + v2
        v2_ref[...] = v2
        vqn_ref[...] = 0.25 * jnp.sum(v2 * v2, axis=1, keepdims=True)
        macc_ref[...] = jnp.zeros_like(macc_ref)

    xt = xt_ref[...]
    dot2 = lax.dot_general(v2_ref[...], xt, (((1,), (0,)), ((), ())),
                           preferred_element_type=jnp.float32)
    idx = jnp.argmin(vqn_ref[...] - dot2, axis=0).astype(jnp.int32)
    idx_ref[...] = idx

    rows = lax.broadcasted_iota(jnp.int32, (K, BLK), 0)
    mask = (rows == idx[None, :]).astype(jnp.float32)
    aug = (lax.broadcasted_iota(jnp.int32, (AUG, BLK), 0) == 0).astype(
        jnp.float32)
    xa_t = jnp.concatenate([xt, aug], axis=0)
    macc_ref[...] += lax.dot_general(mask, xa_t, (((1,), (1,)), ((), ())),
                                     preferred_element_type=jnp.float32)

    @pl.when(i == pl.num_programs(0) - 1)
    def _():
        macc = macc_ref[...]
        csum = macc[:, :D]
        cnt = jnp.sum(macc[:, D:], axis=1)
        new_sum = es_ref[...] * GAMMA + csum * (1.0 - GAMMA)
        new_n = en_ref[...] * GAMMA + cnt * (1.0 - GAMMA)
        ns_ref[...] = new_sum
        nn_ref[...] = new_n
        vq_ref[...] = new_sum / new_n[:, None]


def _main(xt, vqt, ewma_sum, ewma_n):
    return pl.pallas_call(
        _main_body,
        grid=(N // BLK,),
        in_specs=[
            pl.BlockSpec((D, BLK), lambda i: (0, i)),
            pl.BlockSpec((D, K), lambda i: (0, 0)),
            pl.BlockSpec((K, D), lambda i: (0, 0)),
            pl.BlockSpec((K,), lambda i: (0,)),
        ],
        out_specs=[
            pl.BlockSpec((BLK,), lambda i: (i,)),
            pl.BlockSpec((K, D), lambda i: (0, 0)),
            pl.BlockSpec((K, D), lambda i: (0, 0)),
            pl.BlockSpec((K,), lambda i: (0,)),
        ],
        out_shape=[
            jax.ShapeDtypeStruct((N,), jnp.int32),
            jax.ShapeDtypeStruct((K, D), jnp.float32),
            jax.ShapeDtypeStruct((K, D), jnp.float32),
            jax.ShapeDtypeStruct((K,), jnp.float32),
        ],
        scratch_shapes=[pltpu.VMEM((K, D), jnp.float32),
                        pltpu.VMEM((K, 1), jnp.float32),
                        pltpu.VMEM((K, D + AUG), jnp.float32)],
    )(xt, vqt, ewma_sum, ewma_n)


def _sc_body(idx_hbm, vqp_hbm, quant_hbm, idx_v, g0, g1, g2, g3, vqs,
             sem_g, sem_w):
    cid = lax.axis_index("c")
    sid = lax.axis_index("s")
    wid = sid * NC + cid
    base = wid * CHUNK

    pltpu.sync_copy(vqp_hbm.at[pl.ds(sid * (K // NS), K // NS)],
                    vqs.at[pl.ds(sid * (K // NS), K // NS)])
    pltpu.sync_copy(idx_hbm.at[wid], idx_v)
    plsc.subcore_barrier()

    bufs = [g0, g1, g2, g3]
    gets = [pltpu.async_copy(vqs.at[idx_v.at[j]], bufs[j % NBUF], sem_g)
            for j in range(min(NBUF, NSUB))]
    gets += [None] * (NSUB - len(gets))
    writes = []
    for j in range(NSUB):
        gets[j].wait()
        writes.append(
            pltpu.async_copy(bufs[j % NBUF],
                             quant_hbm.at[pl.ds(base + j * SUB, SUB)], sem_w))
        nxt = j + NBUF
        if nxt < NSUB:
            if writes and nxt - NBUF >= 0:
                writes[nxt - NBUF].wait()
                writes[nxt - NBUF] = None
            gets[nxt] = pltpu.async_copy(vqs.at[idx_v.at[nxt]],
                                         bufs[nxt % NBUF], sem_g)
    for w in writes:
        if w is not None:
            w.wait()


@functools.cache
def _sc_stage():
    return functools.partial(
        pl.kernel,
        out_type=jax.ShapeDtypeStruct((N, 2 * D), jnp.float32),
        mesh=plsc.VectorSubcoreMesh(core_axis_name="c", subcore_axis_name="s",
                                    num_cores=NC, num_subcores=NS),
        scratch_types=[
            pltpu.VMEM((NSUB, SUB), jnp.int32),
            pltpu.VMEM((SUB, 2 * D), jnp.float32),
            pltpu.VMEM((SUB, 2 * D), jnp.float32),
            pltpu.VMEM((SUB, 2 * D), jnp.float32),
            pltpu.VMEM((SUB, 2 * D), jnp.float32),
            pltpu.VMEM_SHARED((K, 2 * D), jnp.float32),
            pltpu.SemaphoreType.DMA,
            pltpu.SemaphoreType.DMA,
        ],
    )(_sc_body)


def kernel(x, vq, ewma_centroid_sum, ewma_centroid_n):
    xt = jnp.swapaxes(x, 0, 1)
    vqt = jnp.swapaxes(vq, 0, 1)
    idx, new_vq, new_sum, new_n = _main(xt, vqt, ewma_centroid_sum,
                                        ewma_centroid_n)
    vqp = jnp.concatenate([vq, jnp.zeros((K, D), vq.dtype)], axis=1)
    qpad = _sc_stage()(idx.reshape(NW, NSUB, SUB), vqp)
    quantized = qpad[:, :D]
    return quantized, new_vq, new_sum, new_n


# speedup vs baseline: 2.4977x; 1.0633x over previous
import functools

import jax
import jax.numpy as jnp
from jax import lax
from jax.experimental import pallas as pl
from jax.experimental.pallas import tpu as pltpu
from jax.experimental.pallas import tpu_sc as plsc

D = 64
K = 1024
N = 36864
GAMMA = 0.99

NC = 2
NS = 16
NW = NC * NS
CHUNK = N // NW
SUB = 128
NSUB = CHUNK // SUB
NBUF = 4

BLK = 6144
AUG = 8


def _argmin_body(xt_ref, vqt_ref, idx_ref, v2_ref, vqn_ref):
    @pl.when(pl.program_id(0) == 0)
    def _():
        v2 = jnp.swapaxes(vqt_ref[...], 0, 1)
        v2 = v2 + v2
        v2_ref[...] = v2
        vqn_ref[...] = 0.25 * jnp.sum(v2 * v2, axis=1, keepdims=True)

    dot2 = lax.dot_general(v2_ref[...], xt_ref[...], (((1,), (0,)), ((), ())),
                           preferred_element_type=jnp.float32)
    idx_ref[...] = jnp.argmin(vqn_ref[...] - dot2, axis=0).astype(jnp.int32)


def _compute_indices(xt, vqt):
    return pl.pallas_call(
        _argmin_body,
        grid=(N // BLK,),
        in_specs=[
            pl.BlockSpec((D, BLK), lambda i: (0, i)),
            pl.BlockSpec((D, K), lambda i: (0, 0)),
        ],
        out_specs=pl.BlockSpec((BLK,), lambda i: (i,)),
        out_shape=jax.ShapeDtypeStruct((N,), jnp.int32),
        scratch_shapes=[pltpu.VMEM((K, D), jnp.float32),
                        pltpu.VMEM((K, 1), jnp.float32)],
    )(xt, vqt)


def _stats_body(xt_ref, idx_ref, es_ref, en_ref,
                vq_ref, ns_ref, nn_ref, macc_ref):
    i = pl.program_id(0)

    @pl.when(i == 0)
    def _():
        macc_ref[...] = jnp.zeros_like(macc_ref)

    xt = xt_ref[...]
    idx = idx_ref[...]
    rows = lax.broadcasted_iota(jnp.int32, (K, BLK), 0)
    mask = (rows == idx[None, :]).astype(jnp.float32)
    aug = (lax.broadcasted_iota(jnp.int32, (AUG, BLK), 0) == 0).astype(
        jnp.float32)
    xa_t = jnp.concatenate([xt, aug], axis=0)
    macc_ref[...] += lax.dot_general(mask, xa_t, (((1,), (1,)), ((), ())),
                                     preferred_element_type=jnp.float32)

    @pl.when(i == pl.num_programs(0) - 1)
    def _():
        macc = macc_ref[...]
        csum = macc[:, :D]
        cnt = jnp.sum(macc[:, D:], axis=1)
        new_sum = es_ref[...] * GAMMA + csum * (1.0 - GAMMA)
        new_n = en_ref[...] * GAMMA + cnt * (1.0 - GAMMA)
        ns_ref[...] = new_sum
        nn_ref[...] = new_n
        vq_ref[...] = new_sum / new_n[:, None]


def _stats(xt, idx, ewma_sum, ewma_n):
    return pl.pallas_call(
        _stats_body,
        grid=(N // BLK,),
        in_specs=[
            pl.BlockSpec((D, BLK), lambda i: (0, i)),
            pl.BlockSpec((BLK,), lambda i: (i,)),
            pl.BlockSpec((K, D), lambda i: (0, 0)),
            pl.BlockSpec((K,), lambda i: (0,)),
        ],
        out_specs=[
            pl.BlockSpec((K, D), lambda i: (0, 0)),
            pl.BlockSpec((K, D), lambda i: (0, 0)),
            pl.BlockSpec((K,), lambda i: (0,)),
        ],
        out_shape=[
            jax.ShapeDtypeStruct((K, D), jnp.float32),
            jax.ShapeDtypeStruct((K, D), jnp.float32),
            jax.ShapeDtypeStruct((K,), jnp.float32),
        ],
        scratch_shapes=[pltpu.VMEM((K, D + AUG), jnp.float32)],
    )(xt, idx, ewma_sum, ewma_n)


def _sc_body(idx_hbm, vqp_hbm, quant_hbm, idx_v, g0, g1, g2, g3, vqs,
             sem_g, sem_w):
    cid = lax.axis_index("c")
    sid = lax.axis_index("s")
    wid = sid * NC + cid
    base = wid * CHUNK

    pltpu.sync_copy(vqp_hbm.at[pl.ds(sid * (K // NS), K // NS)],
                    vqs.at[pl.ds(sid * (K // NS), K // NS)])
    pltpu.sync_copy(idx_hbm.at[wid], idx_v)
    plsc.subcore_barrier()

    bufs = [g0, g1, g2, g3]
    gets = [pltpu.async_copy(vqs.at[idx_v.at[j]], bufs[j % NBUF], sem_g)
            for j in range(min(NBUF, NSUB))]
    gets += [None] * (NSUB - len(gets))
    writes = []
    for j in range(NSUB):
        gets[j].wait()
        writes.append(
            pltpu.async_copy(bufs[j % NBUF],
                             quant_hbm.at[pl.ds(base + j * SUB, SUB)], sem_w))
        nxt = j + NBUF
        if nxt < NSUB:
            if writes and nxt - NBUF >= 0:
                writes[nxt - NBUF].wait()
                writes[nxt - NBUF] = None
            gets[nxt] = pltpu.async_copy(vqs.at[idx_v.at[nxt]],
                                         bufs[nxt % NBUF], sem_g)
    for w in writes:
        if w is not None:
            w.wait()


@functools.cache
def _sc_stage():
    return functools.partial(
        pl.kernel,
        out_type=jax.ShapeDtypeStruct((N, 2 * D), jnp.float32),
        mesh=plsc.VectorSubcoreMesh(core_axis_name="c", subcore_axis_name="s",
                                    num_cores=NC, num_subcores=NS),
        scratch_types=[
            pltpu.VMEM((NSUB, SUB), jnp.int32),
            pltpu.VMEM((SUB, 2 * D), jnp.float32),
            pltpu.VMEM((SUB, 2 * D), jnp.float32),
            pltpu.VMEM((SUB, 2 * D), jnp.float32),
            pltpu.VMEM((SUB, 2 * D), jnp.float32),
            pltpu.VMEM_SHARED((K, 2 * D), jnp.float32),
            pltpu.SemaphoreType.DMA,
            pltpu.SemaphoreType.DMA,
        ],
    )(_sc_body)


def kernel(x, vq, ewma_centroid_sum, ewma_centroid_n):
    xt = jnp.swapaxes(x, 0, 1)
    vqt = jnp.swapaxes(vq, 0, 1)
    idx = _compute_indices(xt, vqt)
    vqp = jnp.concatenate([vq, jnp.zeros((K, D), vq.dtype)], axis=1)
    qpad = _sc_stage()(idx.reshape(NW, NSUB, SUB), vqp)
    new_vq, new_sum, new_n = _stats(xt, idx, ewma_centroid_sum,
                                    ewma_centroid_n)
    quantized = qpad[:, :D]
    return quantized, new_vq, new_sum, new_n


# speedup vs baseline: 2.6818x; 1.0737x over previous
import functools

import jax
import jax.numpy as jnp
from jax import lax
from jax.experimental import pallas as pl
from jax.experimental.pallas import tpu as pltpu
from jax.experimental.pallas import tpu_sc as plsc

D = 64
K = 1024
N = 36864
GAMMA = 0.99

NC = 2
NS = 16
NW = NC * NS
CHUNK = N // NW
SUB = 128
NSUB = CHUNK // SUB
NBUF = 4

BLK = 6144
AUG = 8


def _argmin_body(xt_ref, vqt_ref, idx_ref, v2_ref, vqn_ref):
    @pl.when(pl.program_id(0) == 0)
    def _():
        v2 = jnp.swapaxes(vqt_ref[...], 0, 1)
        v2 = v2 + v2
        v2_ref[...] = v2
        vqn_ref[...] = 0.25 * jnp.sum(v2 * v2, axis=1, keepdims=True)

    dot2 = lax.dot_general(v2_ref[...], xt_ref[...], (((1,), (0,)), ((), ())),
                           preferred_element_type=jnp.float32)
    idx_ref[...] = jnp.argmin(vqn_ref[...] - dot2, axis=0).astype(jnp.int32)


def _compute_indices(xt, vqt):
    return pl.pallas_call(
        _argmin_body,
        grid=(N // BLK,),
        in_specs=[
            pl.BlockSpec((D, BLK), lambda i: (0, i)),
            pl.BlockSpec((D, K), lambda i: (0, 0)),
        ],
        out_specs=pl.BlockSpec((BLK,), lambda i: (i,)),
        out_shape=jax.ShapeDtypeStruct((N,), jnp.int32),
        scratch_shapes=[pltpu.VMEM((K, D), jnp.float32),
                        pltpu.VMEM((K, 1), jnp.float32)],
    )(xt, vqt)


def _stats_body(xt_ref, idx_ref, est_ref, en_ref,
                vqt_ref, nst_ref, nn_ref, macc_ref):
    i = pl.program_id(0)

    @pl.when(i == 0)
    def _():
        macc_ref[...] = jnp.zeros_like(macc_ref)

    xt = xt_ref[...]
    idx = idx_ref[...]
    rows = lax.broadcasted_iota(jnp.int32, (K, BLK), 0)
    mask = (rows == idx[None, :]).astype(jnp.bfloat16)
    aug = (lax.broadcasted_iota(jnp.int32, (AUG, BLK), 0) == 0).astype(
        jnp.bfloat16)
    xa_t = jnp.concatenate([xt.astype(jnp.bfloat16), aug], axis=0)
    macc_ref[...] += lax.dot_general(mask, xa_t, (((1,), (1,)), ((), ())),
                                     preferred_element_type=jnp.float32)

    @pl.when(i == pl.num_programs(0) - 1)
    def _():
        macc = macc_ref[...]
        csum_t = jnp.swapaxes(macc[:, :D], 0, 1)
        cnt = jnp.sum(macc[:, D:], axis=1)
        new_sum_t = est_ref[...] * GAMMA + csum_t * (1.0 - GAMMA)
        new_n = en_ref[...] * GAMMA + cnt * (1.0 - GAMMA)
        nst_ref[...] = new_sum_t
        nn_ref[...] = new_n
        vqt_ref[...] = new_sum_t / new_n[None, :]


def _stats(xt, idx, ewma_sum_t, ewma_n):
    return pl.pallas_call(
        _stats_body,
        grid=(N // BLK,),
        in_specs=[
            pl.BlockSpec((D, BLK), lambda i: (0, i)),
            pl.BlockSpec((BLK,), lambda i: (i,)),
            pl.BlockSpec((D, K), lambda i: (0, 0)),
            pl.BlockSpec((K,), lambda i: (0,)),
        ],
        out_specs=[
            pl.BlockSpec((D, K), lambda i: (0, 0)),
            pl.BlockSpec((D, K), lambda i: (0, 0)),
            pl.BlockSpec((K,), lambda i: (0,)),
        ],
        out_shape=[
            jax.ShapeDtypeStruct((D, K), jnp.float32),
            jax.ShapeDtypeStruct((D, K), jnp.float32),
            jax.ShapeDtypeStruct((K,), jnp.float32),
        ],
        scratch_shapes=[pltpu.VMEM((K, D + AUG), jnp.float32)],
    )(xt, idx, ewma_sum_t, ewma_n)


def _sc_body(idx_hbm, vqp_hbm, quant_hbm, idx_v, g0, g1, g2, g3, vqs,
             sem_g, sem_w):
    cid = lax.axis_index("c")
    sid = lax.axis_index("s")
    wid = sid * NC + cid
    base = wid * CHUNK

    pltpu.sync_copy(vqp_hbm.at[pl.ds(sid * (K // NS), K // NS)],
                    vqs.at[pl.ds(sid * (K // NS), K // NS)])
    pltpu.sync_copy(idx_hbm.at[wid], idx_v)
    plsc.subcore_barrier()

    bufs = [g0, g1, g2, g3]
    gets = [pltpu.async_copy(vqs.at[idx_v.at[j]], bufs[j % NBUF], sem_g)
            for j in range(min(NBUF, NSUB))]
    gets += [None] * (NSUB - len(gets))
    writes = []
    for j in range(NSUB):
        gets[j].wait()
        writes.append(
            pltpu.async_copy(bufs[j % NBUF],
                             quant_hbm.at[pl.ds(base + j * SUB, SUB)], sem_w))
        nxt = j + NBUF
        if nxt < NSUB:
            if writes and nxt - NBUF >= 0:
                writes[nxt - NBUF].wait()
                writes[nxt - NBUF] = None
            gets[nxt] = pltpu.async_copy(vqs.at[idx_v.at[nxt]],
                                         bufs[nxt % NBUF], sem_g)
    for w in writes:
        if w is not None:
            w.wait()


@functools.cache
def _sc_stage():
    return functools.partial(
        pl.kernel,
        out_type=jax.ShapeDtypeStruct((N, 2 * D), jnp.float32),
        mesh=plsc.VectorSubcoreMesh(core_axis_name="c", subcore_axis_name="s",
                                    num_cores=NC, num_subcores=NS),
        scratch_types=[
            pltpu.VMEM((NSUB, SUB), jnp.int32),
            pltpu.VMEM((SUB, 2 * D), jnp.float32),
            pltpu.VMEM((SUB, 2 * D), jnp.float32),
            pltpu.VMEM((SUB, 2 * D), jnp.float32),
            pltpu.VMEM((SUB, 2 * D), jnp.float32),
            pltpu.VMEM_SHARED((K, 2 * D), jnp.float32),
            pltpu.SemaphoreType.DMA,
            pltpu.SemaphoreType.DMA,
        ],
    )(_sc_body)


def kernel(x, vq, ewma_centroid_sum, ewma_centroid_n):
    xt = jnp.swapaxes(x, 0, 1)
    vqt = jnp.swapaxes(vq, 0, 1)
    idx = _compute_indices(xt, vqt)
    vqp = jnp.concatenate([vq, jnp.zeros((K, D), vq.dtype)], axis=1)
    qpad = _sc_stage()(idx.reshape(NW, NSUB, SUB), vqp)
    new_vq_t, new_sum_t, new_n = _stats(xt, idx,
                                        jnp.swapaxes(ewma_centroid_sum, 0, 1),
                                        ewma_centroid_n)
    quantized = qpad[:, :D]
    return (quantized, jnp.swapaxes(new_vq_t, 0, 1),
            jnp.swapaxes(new_sum_t, 0, 1), new_n)


# speedup vs baseline: 2.6947x; 1.0048x over previous
import functools

import jax
import jax.numpy as jnp
from jax import lax
from jax.experimental import pallas as pl
from jax.experimental.pallas import tpu as pltpu
from jax.experimental.pallas import tpu_sc as plsc

D = 64
K = 1024
N = 36864
GAMMA = 0.99

NC = 2
NS = 16
NW = NC * NS
CHUNK = N // NW
SUB = 128
NSUB = CHUNK // SUB
NBUF = 4

BLK = 6144
AUG = 8


def _argmin_body(xt_ref, vqt_ref, idx_ref, v2_ref, vqn_ref):
    @pl.when(pl.program_id(0) == 0)
    def _():
        v2 = jnp.swapaxes(vqt_ref[...], 0, 1)
        v2 = v2 + v2
        v2_ref[...] = v2
        vqn_ref[...] = 0.25 * jnp.sum(v2 * v2, axis=1, keepdims=True)

    dot2 = lax.dot_general(v2_ref[...], xt_ref[...], (((1,), (0,)), ((), ())),
                           preferred_element_type=jnp.float32)
    idx_ref[...] = jnp.argmin(vqn_ref[...] - dot2, axis=0).astype(jnp.int32)


def _compute_indices(xt, vqt):
    return pl.pallas_call(
        _argmin_body,
        grid=(N // BLK,),
        in_specs=[
            pl.BlockSpec((D, BLK), lambda i: (0, i)),
            pl.BlockSpec((D, K), lambda i: (0, 0)),
        ],
        out_specs=pl.BlockSpec((BLK,), lambda i: (i,)),
        out_shape=jax.ShapeDtypeStruct((N,), jnp.int32),
        scratch_shapes=[pltpu.VMEM((K, D), jnp.float32),
                        pltpu.VMEM((K, 1), jnp.float32)],
    )(xt, vqt)


def _stats_body(xt_ref, idx_ref, est_ref, en_ref,
                vqt_ref, nst_ref, nn_ref, macc_ref):
    i = pl.program_id(0)

    @pl.when(i == 0)
    def _():
        macc_ref[...] = jnp.zeros_like(macc_ref)

    xt = xt_ref[...]
    idx = idx_ref[...]
    rows = lax.broadcasted_iota(jnp.int32, (K, BLK), 0)
    mask = (rows == idx[None, :]).astype(jnp.bfloat16)
    aug = (lax.broadcasted_iota(jnp.int32, (AUG, BLK), 0) == 0).astype(
        jnp.bfloat16)
    xa_t = jnp.concatenate([xt.astype(jnp.bfloat16), aug], axis=0)
    macc_ref[...] += lax.dot_general(mask, xa_t, (((1,), (1,)), ((), ())),
                                     preferred_element_type=jnp.float32)

    @pl.when(i == pl.num_programs(0) - 1)
    def _():
        macc = macc_ref[...]
        csum_t = jnp.swapaxes(macc[:, :D], 0, 1)
        cnt = jnp.sum(macc[:, D:], axis=1)
        new_sum_t = est_ref[...] * GAMMA + csum_t * (1.0 - GAMMA)
        new_n = en_ref[...] * GAMMA + cnt * (1.0 - GAMMA)
        nst_ref[...] = new_sum_t
        nn_ref[...] = new_n
        vqt_ref[...] = new_sum_t / new_n[None, :]


def _stats(xt, idx, ewma_sum_t, ewma_n):
    return pl.pallas_call(
        _stats_body,
        grid=(N // BLK,),
        in_specs=[
            pl.BlockSpec((D, BLK), lambda i: (0, i)),
            pl.BlockSpec((BLK,), lambda i: (i,)),
            pl.BlockSpec((D, K), lambda i: (0, 0)),
            pl.BlockSpec((K,), lambda i: (0,)),
        ],
        out_specs=[
            pl.BlockSpec((D, K), lambda i: (0, 0)),
            pl.BlockSpec((D, K), lambda i: (0, 0)),
            pl.BlockSpec((K,), lambda i: (0,)),
        ],
        out_shape=[
            jax.ShapeDtypeStruct((D, K), jnp.float32),
            jax.ShapeDtypeStruct((D, K), jnp.float32),
            jax.ShapeDtypeStruct((K,), jnp.float32),
        ],
        scratch_shapes=[pltpu.VMEM((K, D + AUG), jnp.float32)],
    )(xt, idx, ewma_sum_t, ewma_n)


def _sc_body(idx_hbm, vqp_hbm, quant_hbm, idx_v, g0, g1, g2, g3, vqs,
             sem_g, sem_w):
    cid = lax.axis_index("c")
    sid = lax.axis_index("s")
    wid = sid * NC + cid
    base = wid * CHUNK

    pltpu.sync_copy(vqp_hbm.at[pl.ds(sid * (K // NS), K // NS)],
                    vqs.at[pl.ds(sid * (K // NS), K // NS)])
    pltpu.sync_copy(idx_hbm.at[wid], idx_v)
    plsc.subcore_barrier()

    bufs = [g0, g1, g2, g3]
    gets = [pltpu.async_copy(vqs.at[idx_v.at[j]], bufs[j % NBUF], sem_g)
            for j in range(min(NBUF, NSUB))]
    gets += [None] * (NSUB - len(gets))
    writes = []
    for j in range(NSUB):
        gets[j].wait()
        writes.append(
            pltpu.async_copy(bufs[j % NBUF],
                             quant_hbm.at[pl.ds(base + j * SUB, SUB)], sem_w))
        nxt = j + NBUF
        if nxt < NSUB:
            if writes and nxt - NBUF >= 0:
                writes[nxt - NBUF].wait()
                writes[nxt - NBUF] = None
            gets[nxt] = pltpu.async_copy(vqs.at[idx_v.at[nxt]],
                                         bufs[nxt % NBUF], sem_g)
    for w in writes:
        if w is not None:
            w.wait()


@functools.cache
def _sc_stage():
    return functools.partial(
        pl.kernel,
        out_type=jax.ShapeDtypeStruct((N, 2 * D), jnp.float32),
        mesh=plsc.VectorSubcoreMesh(core_axis_name="c", subcore_axis_name="s",
                                    num_cores=NC, num_subcores=NS),
        scratch_types=[
            pltpu.VMEM((NSUB, SUB), jnp.int32),
            pltpu.VMEM((SUB, 2 * D), jnp.float32),
            pltpu.VMEM((SUB, 2 * D), jnp.float32),
            pltpu.VMEM((SUB, 2 * D), jnp.float32),
            pltpu.VMEM((SUB, 2 * D), jnp.float32),
            pltpu.VMEM_SHARED((K, 2 * D), jnp.float32),
            pltpu.SemaphoreType.DMA,
            pltpu.SemaphoreType.DMA,
        ],
    )(_sc_body)


def kernel(x, vq, ewma_centroid_sum, ewma_centroid_n):
    xt = jnp.swapaxes(x, 0, 1)
    vqt = jnp.swapaxes(vq, 0, 1)
    vqp = jnp.concatenate([vq, jnp.zeros((K, D), vq.dtype)], axis=1)
    idx = _compute_indices(xt, vqt)
    qpad = _sc_stage()(idx.reshape(NW, NSUB, SUB), vqp)
    new_vq_t, new_sum_t, new_n = _stats(xt, idx,
                                        jnp.swapaxes(ewma_centroid_sum, 0, 1),
                                        ewma_centroid_n)
    quantized = qpad[:, :D]
    return (quantized, jnp.swapaxes(new_vq_t, 0, 1),
            jnp.swapaxes(new_sum_t, 0, 1), new_n)
